# baseline (speedup 1.0000x reference)
import functools

import jax
import jax.numpy as jnp
from jax import lax
from jax.experimental import pallas as pl
from jax.experimental.pallas import tpu as pltpu
from jax.experimental.pallas import tpu_sc as plsc

N = 10000
E = 320000
H = 128
HH = H // 2
NC = 2
NS = 16
CH = 158
EP = NS * CH * 128
NP = 10240
TPR = NP // NS
BM = 512
DUMP = N

_mesh = plsc.VectorSubcoreMesh(core_axis_name="c", subcore_axis_name="s")


@functools.partial(
    pl.kernel,
    out_type=jax.ShapeDtypeStruct((NC, NP, HH), jnp.float32),
    mesh=_mesh,
    compiler_params=pltpu.CompilerParams(use_tc_tiling_on_sc=False),
    scratch_types=[
        pltpu.VMEM((CH, 128), jnp.int32),
        pltpu.VMEM((CH, 128), jnp.int32),
        pltpu.VMEM((CH, 128), jnp.float32),
        [pltpu.VMEM((128, HH), jnp.float32)] * 2,
        pltpu.VMEM_SHARED((NP, HH), jnp.float32),
        [pltpu.SemaphoreType.DMA] * 2,
        [pltpu.SemaphoreType.DMA] * 2,
    ],
)
def _sc_agg(xt2, srcr, dstr, wr, p_out,
            src_v, dst_v, w_v, bufs, acc, gss, sss):
    c = lax.axis_index("c")
    s = lax.axis_index("s")

    pltpu.sync_copy(srcr.at[s], src_v)
    pltpu.sync_copy(dstr.at[s], dst_v)
    pltpu.sync_copy(wr.at[s], w_v)

    zeros16 = jnp.zeros((16,), jnp.float32)

    def _zrow(i, _):
        for k in range(HH // 16):
            bufs[0][i, pl.ds(k * 16, 16)] = zeros16
        return 0
    lax.fori_loop(0, 128, _zrow, 0)

    def _xidx(i, _):
        v = src_v[i // 8, pl.ds((i % 8) * 16, 16)]
        src_v[i // 8, pl.ds((i % 8) * 16, 16)] = v + v + c
        return 0
    lax.fori_loop(0, CH * 8, _xidx, 0)

    base = s * TPR
    for k in range(TPR // 128):
        pltpu.sync_copy(bufs[0], acc.at[pl.ds(base + k * 128, 128)])
    plsc.subcore_barrier()

    def gstart(b, j):
        pltpu.async_copy(xt2.at[src_v.at[j]], bufs[b], gss[b])

    def gwait(b, j):
        pltpu.make_async_copy(xt2.at[src_v.at[j]], bufs[b], gss[b]).wait()

    def sstart(b, j):
        pltpu.async_copy(bufs[b], acc.at[dst_v.at[j]], sss[b], add=True)

    def swait(b, j):
        pltpu.make_async_copy(bufs[b], acc.at[dst_v.at[j]], sss[b]).wait()

    def scale(b, j):
        bf = bufs[b]
        def _grp(g, _):
            wv16 = w_v[j, pl.ds(g * 16, 16)]
            for i in range(16):
                r = g * 16 + i
                wv = lax.gather(
                    wv16, jnp.full((16, 1), i, jnp.int32),
                    lax.GatherDimensionNumbers(
                        offset_dims=(), collapsed_slice_dims=(0,),
                        start_index_map=(0,)),
                    (1,), mode=lax.GatherScatterMode.PROMISE_IN_BOUNDS)
                for k in range(HH // 16):
                    bf[r, pl.ds(k * 16, 16)] = bf[r, pl.ds(k * 16, 16)] * wv
            return 0
        lax.fori_loop(0, 8, _grp, 0)

    gstart(0, 0)
    gstart(1, 1)

    def _pair(i, _):
        j0 = 2 * i
        j1 = j0 + 1
        for b, j in ((0, j0), (1, j1)):
            gwait(b, j)
            scale(b, j)
            sstart(b, j)
        for b, j in ((0, j0), (1, j1)):
            swait(b, j)
            @pl.when(j + 2 < CH)
            def _(b=b, j=j):
                gstart(b, j + 2)
        return 0
    lax.fori_loop(0, CH // 2, _pair, 0)

    plsc.subcore_barrier()
    pltpu.sync_copy(acc.at[pl.ds(base, TPR)], p_out.at[c, pl.ds(base, TPR)])


@functools.partial(
    pl.kernel,
    out_type=jax.ShapeDtypeStruct((NC, NP, 16), jnp.float32),
    mesh=_mesh,
    compiler_params=pltpu.CompilerParams(use_tc_tiling_on_sc=False),
    scratch_types=[
        pltpu.VMEM((CH, 128), jnp.int32),
        pltpu.VMEM((128, 16), jnp.float32),
        pltpu.VMEM((128, 16), jnp.float32),
        pltpu.VMEM_SHARED((NP, 16), jnp.float32),
        pltpu.SemaphoreType.DMA,
    ],
)
def _sc_cnt(dstr, cnt_out, dst_v, ones_v, zc_v, cnt_sh, sem):
    c = lax.axis_index("c")
    s = lax.axis_index("s")
    pltpu.sync_copy(dstr.at[s], dst_v)

    zeros16 = jnp.zeros((16,), jnp.float32)
    ones16 = jnp.ones((16,), jnp.float32)

    def _fill(i, _):
        ones_v[i, pl.ds(0, 16)] = ones16
        zc_v[i, pl.ds(0, 16)] = zeros16
        return 0
    lax.fori_loop(0, 128, _fill, 0)

    base = s * TPR
    for k in range(TPR // 128):
        pltpu.sync_copy(zc_v, cnt_sh.at[pl.ds(base + k * 128, 128)])
    plsc.subcore_barrier()

    def _pair(i, _):
        j0 = 2 * i
        j1 = j0 + 1
        pltpu.async_copy(ones_v, cnt_sh.at[dst_v.at[j0]], sem, add=True)
        pltpu.async_copy(ones_v, cnt_sh.at[dst_v.at[j1]], sem, add=True)
        pltpu.make_async_copy(ones_v, cnt_sh.at[dst_v.at[j0]], sem).wait()
        pltpu.make_async_copy(ones_v, cnt_sh.at[dst_v.at[j1]], sem).wait()
        return 0
    lax.fori_loop(0, CH // 2, _pair, 0)
    if CH % 2:
        pltpu.sync_copy(ones_v, cnt_sh.at[dst_v.at[CH - 1]], add=True)

    plsc.subcore_barrier()
    pltpu.sync_copy(cnt_sh.at[pl.ds(base, TPR)], cnt_out.at[c, pl.ds(base, TPR)])


def _pre_body(x_ref, wt1_ref, bt1_ref, wt2_ref, bt2_ref, w0_ref, b0_ref,
              t_ref, u0_ref):
    xb = x_ref[...]
    t1 = jnp.maximum(
        jnp.dot(xb, wt1_ref[...], preferred_element_type=jnp.float32)
        + bt1_ref[...], 0.0)
    tv = jnp.sum(t1 * wt2_ref[...], axis=1, keepdims=True) + bt2_ref[...]
    t_ref[...] = jax.nn.sigmoid(tv)
    u0_ref[...] = (jnp.dot(xb, w0_ref[...], preferred_element_type=jnp.float32)
                   + b0_ref[...])


def _w_body(ea_ref, ew_ref, w_ref):
    w_ref[0] = jax.nn.sigmoid(ea_ref[...] * ew_ref[pl.program_id(0), 0])


def _layer_body(p0, p1, u, ct, ew, wt, b, out):
    sl = jax.nn.sigmoid(ew[0, 0])
    deg = 1.0 + ct[...]
    agg = jnp.concatenate([p0[...], p1[...]], axis=1)
    h = jnp.maximum((agg + sl * u[...]) / deg, 0.0)
    out[...] = jnp.dot(h, wt[...], preferred_element_type=jnp.float32) + b[...]


def _final_body(p0, p1, u, ct, ew, t, wl1, a2, bl1, wl2, bl2, out):
    sl = jax.nn.sigmoid(ew[0, 0])
    deg = 1.0 + ct[...]
    agg = jnp.concatenate([p0[...], p1[...]], axis=1)
    h = jnp.maximum((agg + sl * u[...]) / deg, 0.0)
    hc = jnp.maximum(
        jnp.dot(h, wl1[...], preferred_element_type=jnp.float32)
        + t[...] * a2[...] + bl1[...], 0.0)
    out[...] = jax.nn.sigmoid(
        jnp.sum(hc * wl2[...], axis=1, keepdims=True) + bl2[...])


def _full(shape):
    nd = len(shape)
    return pl.BlockSpec(shape, lambda i, _nd=nd: (0,) * _nd)


def _rows(cols):
    return pl.BlockSpec((BM, cols), lambda i: (i, 0))


def kernel(x, edge_index, edge_attr, W0, b0, ew0, W1, b1, ew1, W2, b2, ew2,
           Wt1, bt1, Wt2, bt2, Wl1, bl1, Wl2, bl2):
    f32 = jnp.float32

    xp = jnp.zeros((NP, 256), f32).at[:N, :129].set(x)
    Wt1p = jnp.zeros((256, 64), f32).at[:129, :].set(Wt1.T)
    W0p = jnp.zeros((256, H), f32).at[1:129, :].set(W0.T)

    src = edge_index[0]
    dst = edge_index[1]
    padE = EP - E
    srcp = jnp.concatenate([src, jnp.zeros((padE,), jnp.int32)])
    dstp = jnp.concatenate([dst, jnp.full((padE,), DUMP, jnp.int32)])
    eap = jnp.concatenate([edge_attr[:, 0], jnp.zeros((padE,), f32)])
    srcr = srcp.reshape(NS, CH, 128)
    dstr = dstp.reshape(NS, CH, 128)
    ea2 = eap.reshape(EP // 128, 128)
    ews = jnp.stack([ew0[0], ew1[0], ew2[0]])

    grid = NP // BM

    t_col, u0 = pl.pallas_call(
        _pre_body,
        grid=(grid,),
        in_specs=[_rows(256), _full((256, 64)), _full((1, 64)),
                  _full((1, 64)), _full((1, 1)), _full((256, H)),
                  _full((1, H))],
        out_specs=(_rows(1), _rows(H)),
        out_shape=(jax.ShapeDtypeStruct((NP, 1), f32),
                   jax.ShapeDtypeStruct((NP, H), f32)),
    )(xp, Wt1p, bt1.reshape(1, 64), Wt2, bt2.reshape(1, 1), W0p,
      b0.reshape(1, H))

    w_all = pl.pallas_call(
        _w_body,
        grid=(3,),
        in_specs=[pl.BlockSpec((EP // 128, 128), lambda k: (0, 0)),
                  pl.BlockSpec((3, 1), lambda k: (0, 0))],
        out_specs=pl.BlockSpec((1, EP // 128, 128), lambda k: (k, 0, 0)),
        out_shape=jax.ShapeDtypeStruct((3, EP // 128, 128), f32),
    )(ea2, ews)
    w_all = w_all.reshape(3, NS, CH, 128)


    def layer_combine(p, ct, u, ew, wt, b):
        return pl.pallas_call(
            _layer_body,
            grid=(grid,),
            in_specs=[_rows(HH), _rows(HH), _rows(H), _rows(1),
                      _full((1, 1)), _full((H, H)), _full((1, H))],
            out_specs=_rows(H),
            out_shape=jax.ShapeDtypeStruct((NP, H), f32),
        )(p[0], p[1], u, ct, ew.reshape(1, 1), wt, b.reshape(1, H))

    cntp = _sc_cnt(dstr)
    ct = cntp[0, :, 0:1]

    p = _sc_agg(u0.reshape(2 * NP, HH), srcr, dstr, w_all[0])
    u1 = layer_combine(p, ct, u0, ew0, W1.T, b1)

    p = _sc_agg(u1.reshape(2 * NP, HH), srcr, dstr, w_all[1])
    u2 = layer_combine(p, ct, u1, ew1, W2.T, b2)

    p = _sc_agg(u2.reshape(2 * NP, HH), srcr, dstr, w_all[2])
    out = pl.pallas_call(
        _final_body,
        grid=(grid,),
        in_specs=[_rows(HH), _rows(HH), _rows(H), _rows(1), _full((1, 1)),
                  _rows(1), _full((H, 64)), _full((1, 64)), _full((1, 64)),
                  _full((1, 64)), _full((1, 1))],
        out_specs=_rows(1),
        out_shape=jax.ShapeDtypeStruct((NP, 1), f32),
    )(p[0], p[1], u2, ct, ew2.reshape(1, 1), t_col,
      Wl1[:, :H].T, Wl1[:, H].reshape(1, 64), bl1.reshape(1, 64),
      Wl2, bl2.reshape(1, 1))

    return out[:N, 0]


# speedup vs baseline: 1.5038x; 1.5038x over previous
import functools

import jax
import jax.numpy as jnp
from jax import lax
from jax.experimental import pallas as pl
from jax.experimental.pallas import tpu as pltpu
from jax.experimental.pallas import tpu_sc as plsc

N = 10000
E = 320000
H = 128
HH = H // 2
NC = 2
NS = 16
CH = 158
EP = NS * CH * 128
NP = 10240
TPR = NP // NS
BM = 512
DUMP = N

_mesh = plsc.VectorSubcoreMesh(core_axis_name="c", subcore_axis_name="s")


@functools.partial(
    pl.kernel,
    out_type=jax.ShapeDtypeStruct((NC, NP, HH), jnp.float32),
    mesh=_mesh,
    compiler_params=pltpu.CompilerParams(use_tc_tiling_on_sc=False),
    scratch_types=[
        pltpu.VMEM((CH, 128), jnp.int32),
        pltpu.VMEM((CH, 128), jnp.int32),
        pltpu.VMEM((CH, 128), jnp.float32),
        [pltpu.VMEM((128, HH), jnp.float32)] * 2,
        pltpu.VMEM_SHARED((NP, HH), jnp.float32),
        [pltpu.SemaphoreType.DMA] * 2,
        [pltpu.SemaphoreType.DMA] * 2,
    ],
)
def _sc_agg(xt2, srcr, dstr, wr, p_out,
            src_v, dst_v, w_v, bufs, acc, gss, sss):
    c = lax.axis_index("c")
    s = lax.axis_index("s")

    pltpu.sync_copy(srcr.at[s], src_v)
    pltpu.sync_copy(dstr.at[s], dst_v)
    pltpu.sync_copy(wr.at[s], w_v)

    zeros16 = jnp.zeros((16,), jnp.float32)

    def _zrow(i, _):
        for k in range(HH // 16):
            bufs[0][i, pl.ds(k * 16, 16)] = zeros16
        return 0
    lax.fori_loop(0, 128, _zrow, 0)

    def _xidx(i, _):
        v = src_v[i // 8, pl.ds((i % 8) * 16, 16)]
        src_v[i // 8, pl.ds((i % 8) * 16, 16)] = v + v + c
        return 0
    lax.fori_loop(0, CH * 8, _xidx, 0)

    base = s * TPR
    for k in range(TPR // 128):
        pltpu.sync_copy(bufs[0], acc.at[pl.ds(base + k * 128, 128)])
    plsc.subcore_barrier()

    def gstart(b, j):
        pltpu.async_copy(xt2.at[src_v.at[j]], bufs[b], gss[b])

    def gwait(b, j):
        pltpu.make_async_copy(xt2.at[src_v.at[j]], bufs[b], gss[b]).wait()

    def sstart(b, j):
        pltpu.async_copy(bufs[b], acc.at[dst_v.at[j]], sss[b], add=True)

    def swait(b, j):
        pltpu.make_async_copy(bufs[b], acc.at[dst_v.at[j]], sss[b]).wait()

    def scale(b, j):
        bf = bufs[b]
        def _grp(g, _):
            wv16 = w_v[j, pl.ds(g * 16, 16)]
            for i in range(16):
                r = g * 16 + i
                wv = jnp.full((16,), wv16[i], jnp.float32)
                for k in range(HH // 16):
                    bf[r, pl.ds(k * 16, 16)] = bf[r, pl.ds(k * 16, 16)] * wv
            return 0
        lax.fori_loop(0, 8, _grp, 0)

    gstart(0, 0)
    gstart(1, 1)

    def _pair(i, _):
        j0 = 2 * i
        j1 = j0 + 1
        for b, j in ((0, j0), (1, j1)):
            gwait(b, j)
            scale(b, j)
            sstart(b, j)
        for b, j in ((0, j0), (1, j1)):
            swait(b, j)
            @pl.when(j + 2 < CH)
            def _(b=b, j=j):
                gstart(b, j + 2)
        return 0
    lax.fori_loop(0, CH // 2, _pair, 0)

    plsc.subcore_barrier()
    pltpu.sync_copy(acc.at[pl.ds(base, TPR)], p_out.at[c, pl.ds(base, TPR)])


@functools.partial(
    pl.kernel,
    out_type=jax.ShapeDtypeStruct((NC, NP, 16), jnp.float32),
    mesh=_mesh,
    compiler_params=pltpu.CompilerParams(use_tc_tiling_on_sc=False),
    scratch_types=[
        pltpu.VMEM((CH, 128), jnp.int32),
        pltpu.VMEM((128, 16), jnp.float32),
        pltpu.VMEM((128, 16), jnp.float32),
        pltpu.VMEM_SHARED((NP, 16), jnp.float32),
        pltpu.SemaphoreType.DMA,
    ],
)
def _sc_cnt(dstr, cnt_out, dst_v, ones_v, zc_v, cnt_sh, sem):
    c = lax.axis_index("c")
    s = lax.axis_index("s")
    pltpu.sync_copy(dstr.at[s], dst_v)

    zeros16 = jnp.zeros((16,), jnp.float32)
    ones16 = jnp.ones((16,), jnp.float32)

    def _fill(i, _):
        ones_v[i, pl.ds(0, 16)] = ones16
        zc_v[i, pl.ds(0, 16)] = zeros16
        return 0
    lax.fori_loop(0, 128, _fill, 0)

    base = s * TPR
    for k in range(TPR // 128):
        pltpu.sync_copy(zc_v, cnt_sh.at[pl.ds(base + k * 128, 128)])
    plsc.subcore_barrier()

    def _pair(i, _):
        j0 = 2 * i
        j1 = j0 + 1
        pltpu.async_copy(ones_v, cnt_sh.at[dst_v.at[j0]], sem, add=True)
        pltpu.async_copy(ones_v, cnt_sh.at[dst_v.at[j1]], sem, add=True)
        pltpu.make_async_copy(ones_v, cnt_sh.at[dst_v.at[j0]], sem).wait()
        pltpu.make_async_copy(ones_v, cnt_sh.at[dst_v.at[j1]], sem).wait()
        return 0
    lax.fori_loop(0, CH // 2, _pair, 0)
    if CH % 2:
        pltpu.sync_copy(ones_v, cnt_sh.at[dst_v.at[CH - 1]], add=True)

    plsc.subcore_barrier()
    pltpu.sync_copy(cnt_sh.at[pl.ds(base, TPR)], cnt_out.at[c, pl.ds(base, TPR)])


def _pre_body(x_ref, ea_ref, wt1_ref, bt1_ref, wt2_ref, bt2_ref, w0_ref,
              b0_ref, ews_ref, t_ref, u0_ref, w_ref):
    xb = x_ref[...]
    t1 = jnp.maximum(
        jnp.dot(xb, wt1_ref[...], preferred_element_type=jnp.float32)
        + bt1_ref[...], 0.0)
    tv = jnp.sum(t1 * wt2_ref[...], axis=1, keepdims=True) + bt2_ref[...]
    t_ref[...] = jax.nn.sigmoid(tv)
    u0_ref[...] = (jnp.dot(xb, w0_ref[...], preferred_element_type=jnp.float32)
                   + b0_ref[...])
    for k in range(3):
        w_ref[k] = jax.nn.sigmoid(ea_ref[...] * ews_ref[k, 0])


def _layer_body(p0, p1, u, ct, ew, wt, b, out):
    sl = jax.nn.sigmoid(ew[0, 0])
    deg = 1.0 + ct[...]
    agg = jnp.concatenate([p0[...], p1[...]], axis=1)
    h = jnp.maximum((agg + sl * u[...]) / deg, 0.0)
    out[...] = jnp.dot(h, wt[...], preferred_element_type=jnp.float32) + b[...]


def _final_body(p0, p1, u, ct, ew, t, wl1, a2, bl1, wl2, bl2, out):
    sl = jax.nn.sigmoid(ew[0, 0])
    deg = 1.0 + ct[...]
    agg = jnp.concatenate([p0[...], p1[...]], axis=1)
    h = jnp.maximum((agg + sl * u[...]) / deg, 0.0)
    hc = jnp.maximum(
        jnp.dot(h, wl1[...], preferred_element_type=jnp.float32)
        + t[...] * a2[...] + bl1[...], 0.0)
    out[...] = jax.nn.sigmoid(
        jnp.sum(hc * wl2[...], axis=1, keepdims=True) + bl2[...])


def _full(shape):
    nd = len(shape)
    return pl.BlockSpec(shape, lambda i, _nd=nd: (0,) * _nd)


def _rows(cols):
    return pl.BlockSpec((BM, cols), lambda i: (i, 0))


def kernel(x, edge_index, edge_attr, W0, b0, ew0, W1, b1, ew1, W2, b2, ew2,
           Wt1, bt1, Wt2, bt2, Wl1, bl1, Wl2, bl2):
    f32 = jnp.float32

    xp = jnp.zeros((NP, 256), f32).at[:N, :129].set(x)
    Wt1p = jnp.zeros((256, 64), f32).at[:129, :].set(Wt1.T)
    W0p = jnp.zeros((256, H), f32).at[1:129, :].set(W0.T)

    src = edge_index[0]
    dst = edge_index[1]
    padE = EP - E
    srcp = jnp.concatenate([src, jnp.zeros((padE,), jnp.int32)])
    dstp = jnp.concatenate([dst, jnp.full((padE,), DUMP, jnp.int32)])
    eap = jnp.concatenate([edge_attr[:, 0], jnp.zeros((padE,), f32)])
    srcr = srcp.reshape(NS, CH, 128)
    dstr = dstp.reshape(NS, CH, 128)
    ea2 = eap.reshape(EP // 128, 128)
    ews = jnp.stack([ew0[0], ew1[0], ew2[0]])

    grid = NP // BM

    GP = 4
    ERB = EP // 128 // GP
    t_col, u0, w_all = pl.pallas_call(
        _pre_body,
        grid=(GP,),
        in_specs=[pl.BlockSpec((NP // GP, 256), lambda i: (i, 0)),
                  pl.BlockSpec((ERB, 128), lambda i: (i, 0)),
                  _full((256, 64)), _full((1, 64)),
                  _full((1, 64)), _full((1, 1)), _full((256, H)),
                  _full((1, H)), _full((3, 1))],
        out_specs=(pl.BlockSpec((NP // GP, 1), lambda i: (i, 0)),
                   pl.BlockSpec((NP // GP, H), lambda i: (i, 0)),
                   pl.BlockSpec((3, ERB, 128), lambda i: (0, i, 0))),
        out_shape=(jax.ShapeDtypeStruct((NP, 1), f32),
                   jax.ShapeDtypeStruct((NP, H), f32),
                   jax.ShapeDtypeStruct((3, EP // 128, 128), f32)),
    )(xp, ea2, Wt1p, bt1.reshape(1, 64), Wt2, bt2.reshape(1, 1), W0p,
      b0.reshape(1, H), ews)
    w_all = w_all.reshape(3, NS, CH, 128)


    def layer_combine(p, ct, u, ew, wt, b):
        return pl.pallas_call(
            _layer_body,
            grid=(grid,),
            in_specs=[_rows(HH), _rows(HH), _rows(H), _rows(1),
                      _full((1, 1)), _full((H, H)), _full((1, H))],
            out_specs=_rows(H),
            out_shape=jax.ShapeDtypeStruct((NP, H), f32),
        )(p[0], p[1], u, ct, ew.reshape(1, 1), wt, b.reshape(1, H))

    cntp = _sc_cnt(dstr)
    ct = cntp[0, :, 0:1]

    p = _sc_agg(u0.reshape(2 * NP, HH), srcr, dstr, w_all[0])
    u1 = layer_combine(p, ct, u0, ew0, W1.T, b1)

    p = _sc_agg(u1.reshape(2 * NP, HH), srcr, dstr, w_all[1])
    u2 = layer_combine(p, ct, u1, ew1, W2.T, b2)

    p = _sc_agg(u2.reshape(2 * NP, HH), srcr, dstr, w_all[2])
    out = pl.pallas_call(
        _final_body,
        grid=(grid,),
        in_specs=[_rows(HH), _rows(HH), _rows(H), _rows(1), _full((1, 1)),
                  _rows(1), _full((H, 64)), _full((1, 64)), _full((1, 64)),
                  _full((1, 64)), _full((1, 1))],
        out_specs=_rows(1),
        out_shape=jax.ShapeDtypeStruct((NP, 1), f32),
    )(p[0], p[1], u2, ct, ew2.reshape(1, 1), t_col,
      Wl1[:, :H].T, Wl1[:, H].reshape(1, 64), bl1.reshape(1, 64),
      Wl2, bl2.reshape(1, 1))

    return out[:N, 0]


# speedup vs baseline: 1.5068x; 1.0020x over previous
import functools

import jax
import jax.numpy as jnp
from jax import lax
from jax.experimental import pallas as pl
from jax.experimental.pallas import tpu as pltpu
from jax.experimental.pallas import tpu_sc as plsc

N = 10000
E = 320000
H = 128
HH = H // 2
NC = 2
NS = 16
CH = 158
EP = NS * CH * 128
NP = 10240
TPR = NP // NS
BM = 512
DUMP = N

_mesh = plsc.VectorSubcoreMesh(core_axis_name="c", subcore_axis_name="s")


@functools.partial(
    pl.kernel,
    out_type=jax.ShapeDtypeStruct((NC, NP, HH), jnp.float32),
    mesh=_mesh,
    compiler_params=pltpu.CompilerParams(use_tc_tiling_on_sc=False),
    scratch_types=[
        pltpu.VMEM((CH, 128), jnp.int32),
        pltpu.VMEM((CH, 128), jnp.int32),
        pltpu.VMEM((CH, 128), jnp.float32),
        [pltpu.VMEM((128, HH), jnp.float32)] * 2,
        pltpu.VMEM_SHARED((NP, HH), jnp.float32),
        [pltpu.SemaphoreType.DMA] * 2,
        [pltpu.SemaphoreType.DMA] * 2,
    ],
)
def _sc_agg(xt2, srcr, dstr, wr, dep, p_out,
            src_v, dst_v, w_v, bufs, acc, gss, sss):
    del dep
    c = lax.axis_index("c")
    s = lax.axis_index("s")

    pltpu.sync_copy(srcr.at[s], src_v)
    pltpu.sync_copy(dstr.at[s], dst_v)
    pltpu.sync_copy(wr.at[s], w_v)

    zeros16 = jnp.zeros((16,), jnp.float32)

    def _zrow(i, _):
        for k in range(HH // 16):
            bufs[0][i, pl.ds(k * 16, 16)] = zeros16
        return 0
    lax.fori_loop(0, 128, _zrow, 0)

    def _xidx(i, _):
        v = src_v[i // 8, pl.ds((i % 8) * 16, 16)]
        src_v[i // 8, pl.ds((i % 8) * 16, 16)] = v + v + c
        return 0
    lax.fori_loop(0, CH * 8, _xidx, 0)

    base = s * TPR
    for k in range(TPR // 128):
        pltpu.sync_copy(bufs[0], acc.at[pl.ds(base + k * 128, 128)])
    plsc.subcore_barrier()

    def gstart(b, j):
        pltpu.async_copy(xt2.at[src_v.at[j]], bufs[b], gss[b])

    def gwait(b, j):
        pltpu.make_async_copy(xt2.at[src_v.at[j]], bufs[b], gss[b]).wait()

    def sstart(b, j):
        pltpu.async_copy(bufs[b], acc.at[dst_v.at[j]], sss[b], add=True)

    def swait(b, j):
        pltpu.make_async_copy(bufs[b], acc.at[dst_v.at[j]], sss[b]).wait()

    def scale(b, j):
        bf = bufs[b]
        def _grp(g, _):
            wv16 = w_v[j, pl.ds(g * 16, 16)]
            for i in range(16):
                r = g * 16 + i
                wv = jnp.full((16,), wv16[i], jnp.float32)
                for k in range(HH // 16):
                    bf[r, pl.ds(k * 16, 16)] = bf[r, pl.ds(k * 16, 16)] * wv
            return 0
        lax.fori_loop(0, 8, _grp, 0)

    gstart(0, 0)
    gstart(1, 1)

    def _pair(i, _):
        j0 = 2 * i
        j1 = j0 + 1
        for b, j in ((0, j0), (1, j1)):
            gwait(b, j)
            scale(b, j)
            sstart(b, j)
        for b, j in ((0, j0), (1, j1)):
            swait(b, j)
            @pl.when(j + 2 < CH)
            def _(b=b, j=j):
                gstart(b, j + 2)
        return 0
    lax.fori_loop(0, CH // 2, _pair, 0)

    plsc.subcore_barrier()
    pltpu.sync_copy(acc.at[pl.ds(base, TPR)], p_out.at[c, pl.ds(base, TPR)])


@functools.partial(
    pl.kernel,
    out_type=jax.ShapeDtypeStruct((NC, NP, 16), jnp.float32),
    mesh=_mesh,
    compiler_params=pltpu.CompilerParams(use_tc_tiling_on_sc=False),
    scratch_types=[
        pltpu.VMEM((CH, 128), jnp.int32),
        pltpu.VMEM((128, 16), jnp.float32),
        pltpu.VMEM((128, 16), jnp.float32),
        pltpu.VMEM_SHARED((NP, 16), jnp.float32),
        pltpu.SemaphoreType.DMA,
    ],
)
def _sc_cnt(dstr, cnt_out, dst_v, ones_v, zc_v, cnt_sh, sem):
    c = lax.axis_index("c")
    s = lax.axis_index("s")
    pltpu.sync_copy(dstr.at[s], dst_v)

    zeros16 = jnp.zeros((16,), jnp.float32)
    ones16 = jnp.ones((16,), jnp.float32)

    def _fill(i, _):
        ones_v[i, pl.ds(0, 16)] = ones16
        zc_v[i, pl.ds(0, 16)] = zeros16
        return 0
    lax.fori_loop(0, 128, _fill, 0)

    base = s * TPR
    for k in range(TPR // 128):
        pltpu.sync_copy(zc_v, cnt_sh.at[pl.ds(base + k * 128, 128)])
    plsc.subcore_barrier()

    def _pair(i, _):
        j0 = 2 * i
        j1 = j0 + 1
        pltpu.async_copy(ones_v, cnt_sh.at[dst_v.at[j0]], sem, add=True)
        pltpu.async_copy(ones_v, cnt_sh.at[dst_v.at[j1]], sem, add=True)
        pltpu.make_async_copy(ones_v, cnt_sh.at[dst_v.at[j0]], sem).wait()
        pltpu.make_async_copy(ones_v, cnt_sh.at[dst_v.at[j1]], sem).wait()
        return 0
    lax.fori_loop(0, CH // 2, _pair, 0)
    if CH % 2:
        pltpu.sync_copy(ones_v, cnt_sh.at[dst_v.at[CH - 1]], add=True)

    plsc.subcore_barrier()
    pltpu.sync_copy(cnt_sh.at[pl.ds(base, TPR)], cnt_out.at[c, pl.ds(base, TPR)])


def _pre_body(x_ref, ea_ref, wt1_ref, bt1_ref, wt2_ref, bt2_ref, w0_ref,
              b0_ref, ews_ref, t_ref, u0_ref, w_ref):
    xb = x_ref[...]
    t1 = jnp.maximum(
        jnp.dot(xb, wt1_ref[...], preferred_element_type=jnp.float32)
        + bt1_ref[...], 0.0)
    tv = jnp.sum(t1 * wt2_ref[...], axis=1, keepdims=True) + bt2_ref[...]
    t_ref[...] = jax.nn.sigmoid(tv)
    u0_ref[...] = (jnp.dot(xb, w0_ref[...], preferred_element_type=jnp.float32)
                   + b0_ref[...])
    for k in range(3):
        w_ref[k] = jax.nn.sigmoid(ea_ref[...] * ews_ref[k, 0])


def _layer_body(p0, p1, u, ct, ew, wt, b, out):
    sl = jax.nn.sigmoid(ew[0, 0])
    deg = 1.0 + ct[...]
    agg = jnp.concatenate([p0[...], p1[...]], axis=1)
    h = jnp.maximum((agg + sl * u[...]) / deg, 0.0)
    out[...] = jnp.dot(h, wt[...], preferred_element_type=jnp.float32) + b[...]


def _final_body(p0, p1, u, ct, ew, t, wl1, a2, bl1, wl2, bl2, out):
    sl = jax.nn.sigmoid(ew[0, 0])
    deg = 1.0 + ct[...]
    agg = jnp.concatenate([p0[...], p1[...]], axis=1)
    h = jnp.maximum((agg + sl * u[...]) / deg, 0.0)
    hc = jnp.maximum(
        jnp.dot(h, wl1[...], preferred_element_type=jnp.float32)
        + t[...] * a2[...] + bl1[...], 0.0)
    out[...] = jax.nn.sigmoid(
        jnp.sum(hc * wl2[...], axis=1, keepdims=True) + bl2[...])


def _full(shape):
    nd = len(shape)
    return pl.BlockSpec(shape, lambda i, _nd=nd: (0,) * _nd)


def _rows(cols):
    return pl.BlockSpec((BM, cols), lambda i: (i, 0))


def kernel(x, edge_index, edge_attr, W0, b0, ew0, W1, b1, ew1, W2, b2, ew2,
           Wt1, bt1, Wt2, bt2, Wl1, bl1, Wl2, bl2):
    f32 = jnp.float32

    xp = jnp.zeros((NP, 256), f32).at[:N, :129].set(x)
    Wt1p = jnp.zeros((256, 64), f32).at[:129, :].set(Wt1.T)
    W0p = jnp.zeros((256, H), f32).at[1:129, :].set(W0.T)

    src = edge_index[0]
    dst = edge_index[1]
    padE = EP - E
    srcp = jnp.concatenate([src, jnp.zeros((padE,), jnp.int32)])
    dstp = jnp.concatenate([dst, jnp.full((padE,), DUMP, jnp.int32)])
    eap = jnp.concatenate([edge_attr[:, 0], jnp.zeros((padE,), f32)])
    srcr = srcp.reshape(NS, CH, 128)
    dstr = dstp.reshape(NS, CH, 128)
    ea2 = eap.reshape(EP // 128, 128)
    ews = jnp.stack([ew0[0], ew1[0], ew2[0]])

    grid = NP // BM

    GP = 4
    ERB = EP // 128 // GP
    t_col, u0, w_all = pl.pallas_call(
        _pre_body,
        grid=(GP,),
        in_specs=[pl.BlockSpec((NP // GP, 256), lambda i: (i, 0)),
                  pl.BlockSpec((ERB, 128), lambda i: (i, 0)),
                  _full((256, 64)), _full((1, 64)),
                  _full((1, 64)), _full((1, 1)), _full((256, H)),
                  _full((1, H)), _full((3, 1))],
        out_specs=(pl.BlockSpec((NP // GP, 1), lambda i: (i, 0)),
                   pl.BlockSpec((NP // GP, H), lambda i: (i, 0)),
                   pl.BlockSpec((3, ERB, 128), lambda i: (0, i, 0))),
        out_shape=(jax.ShapeDtypeStruct((NP, 1), f32),
                   jax.ShapeDtypeStruct((NP, H), f32),
                   jax.ShapeDtypeStruct((3, EP // 128, 128), f32)),
    )(xp, ea2, Wt1p, bt1.reshape(1, 64), Wt2, bt2.reshape(1, 1), W0p,
      b0.reshape(1, H), ews)
    w_all = w_all.reshape(3, NS, CH, 128)


    def layer_combine(p, ct, u, ew, wt, b):
        return pl.pallas_call(
            _layer_body,
            grid=(grid,),
            in_specs=[_rows(HH), _rows(HH), _rows(H), _rows(1),
                      _full((1, 1)), _full((H, H)), _full((1, H))],
            out_specs=_rows(H),
            out_shape=jax.ShapeDtypeStruct((NP, H), f32),
        )(p[0], p[1], u, ct, ew.reshape(1, 1), wt, b.reshape(1, H))

    cntp = _sc_cnt(dstr)
    ct = cntp[0, :, 0:1]

    p = _sc_agg(u0.reshape(2 * NP, HH), srcr, dstr, w_all[0], cntp)
    u1 = layer_combine(p, ct, u0, ew0, W1.T, b1)

    p = _sc_agg(u1.reshape(2 * NP, HH), srcr, dstr, w_all[1], cntp)
    u2 = layer_combine(p, ct, u1, ew1, W2.T, b2)

    p = _sc_agg(u2.reshape(2 * NP, HH), srcr, dstr, w_all[2], cntp)
    out = pl.pallas_call(
        _final_body,
        grid=(grid,),
        in_specs=[_rows(HH), _rows(HH), _rows(H), _rows(1), _full((1, 1)),
                  _rows(1), _full((H, 64)), _full((1, 64)), _full((1, 64)),
                  _full((1, 64)), _full((1, 1))],
        out_specs=_rows(1),
        out_shape=jax.ShapeDtypeStruct((NP, 1), f32),
    )(p[0], p[1], u2, ct, ew2.reshape(1, 1), t_col,
      Wl1[:, :H].T, Wl1[:, H].reshape(1, 64), bl1.reshape(1, 64),
      Wl2, bl2.reshape(1, 1))

    return out[:N, 0]
